# baseline (speedup 1.0000x reference)
import jax
import jax.numpy as jnp
from jax import lax
from jax.experimental import pallas as pl
from jax.experimental.pallas import tpu as pltpu
from jax.experimental.pallas import tpu_sc as plsc

B, N, E, D = 4, 10000, 320000, 128
NC, NS, L = 2, 16, 16
K = 80
EPT = E // NS
CHUNKS = EPT // K
NBUF = 4
STEPS = CHUNKS // NBUF + 1
RT = 624
TAIL = N - NS * RT
BR = 16


def _gcn_kernel(idx_hbm, ann_hbm, bias_hbm, out_hbm,
                acc, bias_buf, rows0, rows1, rows2, rows3,
                ib0, ib1, ib2, ib3, dc0, dc1, dc2, dc3,
                sx0, sx1, sx2, sx3, sg0, sg1, sg2, sg3,
                ss0, ss1, ss2, ss3):
    rows = [rows0, rows1, rows2, rows3]
    idxb = [ib0, ib1, ib2, ib3]
    dcur = [dc0, dc1, dc2, dc3]
    sem_x = [sx0, sx1, sx2, sx3]
    sem_g = [sg0, sg1, sg2, sg3]
    sem_s = [ss0, ss1, ss2, ss3]
    cid = lax.axis_index("c")
    tid = lax.axis_index("s")

    pltpu.sync_copy(bias_hbm, bias_buf.at[pl.ds(0, 1)])
    bv = [bias_buf[0, pl.ds(d * L, L)] for d in range(D // L)]
    for r in range(1, BR):
        for d in range(D // L):
            bias_buf[r, pl.ds(d * L, L)] = bv[d]

    def issue_idx(ebase, c, p):
        off = (ebase + c) * (3 * K)
        pltpu.async_copy(idx_hbm.at[pl.ds(off, 3 * K)], idxb[p], sem_x[p])

    def wait_idx(p):
        pltpu.make_async_copy(idx_hbm.at[pl.ds(0, 3 * K)], idxb[p],
                              sem_x[p]).wait()

    def issue_gather(b, p):
        pltpu.async_copy(ann_hbm.at[b].at[idxb[p].at[pl.ds(0, K)]],
                         rows[p], sem_g[p])

    def wait_gather(b, p):
        pltpu.make_async_copy(ann_hbm.at[b].at[idxb[p].at[pl.ds(0, K)]],
                              rows[p], sem_g[p]).wait()

    def issue_scatter(p):
        pltpu.async_copy(rows[p], acc.at[dcur[p]], sem_s[p], add=True)

    def wait_scatter(p):
        pltpu.make_async_copy(rows[p], acc.at[dcur[p]], sem_s[p]).wait()

    def stage(p):
        for g in range(K // L):
            dcur[p][pl.ds(g * L, L)] = idxb[p][pl.ds(K + g * L, L)]
        return [lax.bitcast_convert_type(idxb[p][pl.ds(2 * K + g * L, L)],
                                         jnp.float32)
                for g in range(K // L)]

    def scale(p, evgs):
        r = rows[p]
        for g in range(K // L):
            for i in range(L):
                e = g * L + i
                evs = lax.gather(
                    evgs[g], jnp.full((L, 1), i, jnp.int32),
                    lax.GatherDimensionNumbers(
                        offset_dims=(), collapsed_slice_dims=(0,),
                        start_index_map=(0,)),
                    slice_sizes=(1,),
                    mode=lax.GatherScatterMode.PROMISE_IN_BOUNDS)
                for d in range(D // L):
                    sl = pl.ds(d * L, L)
                    r[e, sl] = r[e, sl] * evs

    for bb in range(B // NC):
        b = cid * (B // NC) + bb
        ebase = (b * NS + tid) * CHUNKS

        row0 = tid * RT
        for r in range(0, RT, BR):
            pltpu.sync_copy(bias_buf, acc.at[pl.ds(row0 + r, BR)])

        @pl.when(tid == 0)
        def _():
            pltpu.sync_copy(bias_buf.at[pl.ds(0, TAIL)],
                            acc.at[pl.ds(NS * RT, TAIL)])

        plsc.subcore_barrier()

        issue_idx(ebase, 0, 0)
        issue_idx(ebase, 1, 1)
        issue_idx(ebase, 2, 2)
        wait_idx(0)
        issue_gather(b, 0)
        wait_idx(1)
        issue_gather(b, 1)

        def loop_body(t, carry):
            not_last = t < STEPS - 1
            for u in range(NBUF):
                c = NBUF * t + u

                if u < 2:
                    @pl.when(t > 0)
                    def _():
                        wait_scatter((u + 2) % NBUF)
                else:
                    wait_scatter((u + 2) % NBUF)

                @pl.when(not_last)
                def _():
                    wait_idx((u + 2) % NBUF)
                    issue_gather(b, (u + 2) % NBUF)

                def _body():
                    wait_gather(b, u)
                    evgs = stage(u)

                    if u == NBUF - 1:
                        @pl.when(t < STEPS - 2)
                        def _():
                            issue_idx(ebase, c + 3, (u + 3) % NBUF)
                    else:
                        @pl.when(not_last)
                        def _():
                            issue_idx(ebase, c + 3, (u + 3) % NBUF)

                    scale(u, evgs)
                    issue_scatter(u)
                if u < 2:
                    _body()
                else:
                    pl.when(not_last)(_body)
            return carry

        lax.fori_loop(0, STEPS, loop_body, 0)

        plsc.subcore_barrier()
        pltpu.sync_copy(acc.at[pl.ds(row0, RT)],
                        out_hbm.at[b, pl.ds(row0, RT)])

        @pl.when(tid == 0)
        def _():
            pltpu.sync_copy(acc.at[pl.ds(NS * RT, TAIL)],
                            out_hbm.at[b, pl.ds(NS * RT, TAIL)])

        plsc.subcore_barrier()


@jax.jit
def _gcn(idx, ann, bias_row):
    mesh = plsc.VectorSubcoreMesh(core_axis_name="c", subcore_axis_name="s")
    k = pl.kernel(
        _gcn_kernel,
        out_type=jax.ShapeDtypeStruct((B, N, D), jnp.float32),
        mesh=mesh,
        scratch_types=(
            [pltpu.VMEM_SHARED((N, D), jnp.float32),
             pltpu.VMEM((BR, D), jnp.float32)]
            + [pltpu.VMEM((K, D), jnp.float32) for _ in range(NBUF)]
            + [pltpu.VMEM((3 * K,), jnp.int32) for _ in range(NBUF)]
            + [pltpu.VMEM((K,), jnp.int32) for _ in range(NBUF)]
            + [pltpu.SemaphoreType.DMA for _ in range(3 * NBUF)]
        ),
    )
    return k(idx, ann, bias_row)


def kernel(edge_index, edge_vals, annotations, bias):
    dst = edge_index[:, 0, :].reshape(B, NS, CHUNKS, K)
    src = edge_index[:, 1, :].reshape(B, NS, CHUNKS, K)
    ev = lax.bitcast_convert_type(edge_vals, jnp.int32).reshape(
        B, NS, CHUNKS, K)
    idx = jnp.stack([src, dst, ev], axis=3).reshape(B * NS * CHUNKS * 3 * K)
    bias_row = bias.reshape(1, D)
    return _gcn(idx, annotations, bias_row)


# speedup vs baseline: 1.1052x; 1.1052x over previous
import jax
import jax.numpy as jnp
from jax import lax
from jax.experimental import pallas as pl
from jax.experimental.pallas import tpu as pltpu
from jax.experimental.pallas import tpu_sc as plsc

B, N, E, D = 4, 10000, 320000, 128
NC, NS, L = 2, 16, 16
K = 80
EPT = E // NS
CHUNKS = EPT // K
STEPS = CHUNKS // 2
RT = 624
TAIL = N - NS * RT
BR = 16

def _gcn_kernel(idx_hbm, ann_hbm, bias_hbm, out_hbm,
                acc, bias_buf, rin0, rin1, rout0, rout1,
                ib0, ib1, dc0, dc1,
                sx0, sx1, sg0, sg1, ss0, ss1):
    rin = [rin0, rin1]
    rout = [rout0, rout1]
    idxb = [ib0, ib1]
    dcur = [dc0, dc1]
    sem_x = [sx0, sx1]
    sem_g = [sg0, sg1]
    sem_s = [ss0, ss1]
    cid = lax.axis_index("c")
    tid = lax.axis_index("s")

    pltpu.sync_copy(bias_hbm, bias_buf.at[pl.ds(0, 1)])
    bv = [bias_buf[0, pl.ds(d * L, L)] for d in range(D // L)]
    for r in range(1, BR):
        for d in range(D // L):
            bias_buf[r, pl.ds(d * L, L)] = bv[d]

    def issue_idx(ebase, c, p):
        off = (ebase + c) * (3 * K)
        pltpu.async_copy(idx_hbm.at[pl.ds(off, 3 * K)], idxb[p], sem_x[p])

    def wait_idx(p):
        pltpu.make_async_copy(idx_hbm.at[pl.ds(0, 3 * K)], idxb[p],
                              sem_x[p]).wait()

    def issue_gather(b, p):
        pltpu.async_copy(ann_hbm.at[b].at[idxb[p].at[pl.ds(0, K)]],
                         rin[p], sem_g[p])

    def wait_gather(b, p):
        pltpu.make_async_copy(ann_hbm.at[b].at[idxb[p].at[pl.ds(0, K)]],
                              rin[p], sem_g[p]).wait()

    def issue_scatter(p):
        pltpu.async_copy(rout[p], acc.at[dcur[p]], sem_s[p], add=True)

    def wait_scatter(p):
        pltpu.make_async_copy(rout[p], acc.at[dcur[p]], sem_s[p]).wait()

    def stage(p):
        for g in range(K // L):
            dcur[p][pl.ds(g * L, L)] = idxb[p][pl.ds(K + g * L, L)]
        return [lax.bitcast_convert_type(idxb[p][pl.ds(2 * K + g * L, L)],
                                         jnp.float32)
                for g in range(K // L)]

    def scale(p, evgs):
        ri = rin[p]
        ro = rout[p]
        for g in range(K // L):
            for i in range(L):
                e = g * L + i
                evs = lax.gather(
                    evgs[g], jnp.full((L, 1), i, jnp.int32),
                    lax.GatherDimensionNumbers(
                        offset_dims=(), collapsed_slice_dims=(0,),
                        start_index_map=(0,)),
                    slice_sizes=(1,),
                    mode=lax.GatherScatterMode.PROMISE_IN_BOUNDS)
                for d in range(D // L):
                    sl = pl.ds(d * L, L)
                    ro[e, sl] = ri[e, sl] * evs

    for bb in range(B // NC):
        b = cid * (B // NC) + bb
        ebase = (b * NS + tid) * CHUNKS

        row0 = tid * RT
        for r in range(0, RT, BR):
            pltpu.sync_copy(bias_buf, acc.at[pl.ds(row0 + r, BR)])

        @pl.when(tid == 0)
        def _():
            pltpu.sync_copy(bias_buf.at[pl.ds(0, TAIL)],
                            acc.at[pl.ds(NS * RT, TAIL)])

        plsc.subcore_barrier()

        issue_idx(ebase, 0, 0)
        wait_idx(0)
        issue_gather(b, 0)
        issue_idx(ebase, 1, 1)

        def loop_body(t, carry):
            for u in range(2):
                c = 2 * t + u

                @pl.when(t > 0)
                def _():
                    wait_scatter(u)

                def _wg():
                    wait_idx((u + 1) % 2)
                    issue_gather(b, (u + 1) % 2)
                if u == 0:
                    _wg()
                else:
                    pl.when(t < STEPS - 1)(_wg)

                wait_gather(b, u)
                evgs = stage(u)

                @pl.when(t < STEPS - 1)
                def _():
                    issue_idx(ebase, c + 2, u)

                scale(u, evgs)
                issue_scatter(u)
            return carry

        lax.fori_loop(0, STEPS, loop_body, 0)

        wait_scatter(0)
        wait_scatter(1)

        plsc.subcore_barrier()
        pltpu.sync_copy(acc.at[pl.ds(row0, RT)],
                        out_hbm.at[b, pl.ds(row0, RT)])

        @pl.when(tid == 0)
        def _():
            pltpu.sync_copy(acc.at[pl.ds(NS * RT, TAIL)],
                            out_hbm.at[b, pl.ds(NS * RT, TAIL)])

        plsc.subcore_barrier()


@jax.jit
def _gcn(idx, ann, bias_row):
    mesh = plsc.VectorSubcoreMesh(core_axis_name="c", subcore_axis_name="s")
    k = pl.kernel(
        _gcn_kernel,
        out_type=jax.ShapeDtypeStruct((B, N, D), jnp.float32),
        mesh=mesh,
        scratch_types=(
            [pltpu.VMEM_SHARED((N, D), jnp.float32),
             pltpu.VMEM((BR, D), jnp.float32)]
            + [pltpu.VMEM((K, D), jnp.float32) for _ in range(2)]
            + [pltpu.VMEM((K, D), jnp.float32) for _ in range(2)]
            + [pltpu.VMEM((3 * K,), jnp.int32) for _ in range(2)]
            + [pltpu.VMEM((K,), jnp.int32) for _ in range(2)]
            + [pltpu.SemaphoreType.DMA for _ in range(6)]
        ),
    )
    return k(idx, ann, bias_row)


def kernel(edge_index, edge_vals, annotations, bias):
    dst = edge_index[:, 0, :].reshape(B, NS, CHUNKS, K)
    src = edge_index[:, 1, :].reshape(B, NS, CHUNKS, K)
    ev = lax.bitcast_convert_type(edge_vals, jnp.int32).reshape(
        B, NS, CHUNKS, K)
    idx = jnp.stack([src, dst, ev], axis=3).reshape(B * NS * CHUNKS * 3 * K)
    bias_row = bias.reshape(1, D)
    return _gcn(idx, annotations, bias_row)


# speedup vs baseline: 1.1066x; 1.0013x over previous
import jax
import jax.numpy as jnp
from jax import lax
from jax.experimental import pallas as pl
from jax.experimental.pallas import tpu as pltpu
from jax.experimental.pallas import tpu_sc as plsc

B, N, E, D = 4, 10000, 320000, 128
NC, NS, L = 2, 16, 16
K = 80
EPT = E // NS
CHUNKS = EPT // K
STEPS = CHUNKS // 2
RT = 624
TAIL = N - NS * RT
BR = 16

def _gcn_kernel(idx_hbm, ann_hbm, bias_hbm, out_hbm,
                acc, bias_buf, rin0, rin1, rout0, rout1,
                ib0, ib1, dc0, dc1,
                sx0, sx1, sg0, sg1, ss0, ss1):
    rin = [rin0, rin1]
    rout = [rout0, rout1]
    idxb = [ib0, ib1]
    dcur = [dc0, dc1]
    sem_x = [sx0, sx1]
    sem_g = [sg0, sg1]
    sem_s = [ss0, ss1]
    cid = lax.axis_index("c")
    tid = lax.axis_index("s")

    pltpu.sync_copy(bias_hbm, bias_buf.at[pl.ds(0, 1)])
    bv = [bias_buf[0, pl.ds(d * L, L)] for d in range(D // L)]
    for r in range(1, BR):
        for d in range(D // L):
            bias_buf[r, pl.ds(d * L, L)] = bv[d]

    def issue_idx(ebase, c, p):
        off = (ebase + c) * (3 * K)
        pltpu.async_copy(idx_hbm.at[pl.ds(off, 3 * K)], idxb[p], sem_x[p])

    def wait_idx(p):
        pltpu.make_async_copy(idx_hbm.at[pl.ds(0, 3 * K)], idxb[p],
                              sem_x[p]).wait()

    def issue_gather(b, p):
        pltpu.async_copy(ann_hbm.at[b].at[idxb[p].at[pl.ds(0, K)]],
                         rin[p], sem_g[p])

    def wait_gather(b, p):
        pltpu.make_async_copy(ann_hbm.at[b].at[idxb[p].at[pl.ds(0, K)]],
                              rin[p], sem_g[p]).wait()

    def issue_scatter(p):
        pltpu.async_copy(rout[p], acc.at[dcur[p]], sem_s[p], add=True)

    def wait_scatter(p):
        pltpu.make_async_copy(rout[p], acc.at[dcur[p]], sem_s[p]).wait()

    def stage(p):
        for g in range(K // L):
            dcur[p][pl.ds(g * L, L)] = idxb[p][pl.ds(K + g * L, L)]
        return [lax.bitcast_convert_type(idxb[p][pl.ds(2 * K + g * L, L)],
                                         jnp.float32)
                for g in range(K // L)]

    def scale(p, evgs):
        ri = rin[p]
        ro = rout[p]
        for g in range(K // L):
            for i in range(L):
                e = g * L + i
                evs = lax.gather(
                    evgs[g], jnp.full((L, 1), i, jnp.int32),
                    lax.GatherDimensionNumbers(
                        offset_dims=(), collapsed_slice_dims=(0,),
                        start_index_map=(0,)),
                    slice_sizes=(1,),
                    mode=lax.GatherScatterMode.PROMISE_IN_BOUNDS)
                for d in range(D // L):
                    sl = pl.ds(d * L, L)
                    ro[e, sl] = ri[e, sl] * evs

    for bb in range(B // NC):
        b = cid * (B // NC) + bb
        ebase = (b * NS + tid) * CHUNKS

        row0 = tid * RT
        for r in range(0, RT, BR):
            pltpu.async_copy(bias_buf, acc.at[pl.ds(row0 + r, BR)], sx0)

        @pl.when(tid == 0)
        def _():
            pltpu.async_copy(bias_buf.at[pl.ds(0, TAIL)],
                             acc.at[pl.ds(NS * RT, TAIL)], sx0)

        for r in range(0, RT, BR):
            pltpu.make_async_copy(bias_buf, acc.at[pl.ds(row0 + r, BR)],
                                  sx0).wait()

        @pl.when(tid == 0)
        def _():
            pltpu.make_async_copy(bias_buf.at[pl.ds(0, TAIL)],
                                  acc.at[pl.ds(NS * RT, TAIL)], sx0).wait()

        plsc.subcore_barrier()

        issue_idx(ebase, 0, 0)
        wait_idx(0)
        issue_gather(b, 0)
        issue_idx(ebase, 1, 1)

        def loop_body(t, carry):
            for u in range(2):
                c = 2 * t + u

                @pl.when(t > 0)
                def _():
                    wait_scatter(u)

                def _wg():
                    wait_idx((u + 1) % 2)
                    issue_gather(b, (u + 1) % 2)
                if u == 0:
                    _wg()
                else:
                    pl.when(t < STEPS - 1)(_wg)

                wait_gather(b, u)
                evgs = stage(u)

                @pl.when(t < STEPS - 1)
                def _():
                    issue_idx(ebase, c + 2, u)

                scale(u, evgs)
                issue_scatter(u)
            return carry

        lax.fori_loop(0, STEPS, loop_body, 0)

        wait_scatter(0)
        wait_scatter(1)

        plsc.subcore_barrier()
        pltpu.sync_copy(acc.at[pl.ds(row0, RT)],
                        out_hbm.at[b, pl.ds(row0, RT)])

        @pl.when(tid == 0)
        def _():
            pltpu.sync_copy(acc.at[pl.ds(NS * RT, TAIL)],
                            out_hbm.at[b, pl.ds(NS * RT, TAIL)])

        plsc.subcore_barrier()


@jax.jit
def _gcn(idx, ann, bias_row):
    mesh = plsc.VectorSubcoreMesh(core_axis_name="c", subcore_axis_name="s")
    k = pl.kernel(
        _gcn_kernel,
        out_type=jax.ShapeDtypeStruct((B, N, D), jnp.float32),
        mesh=mesh,
        scratch_types=(
            [pltpu.VMEM_SHARED((N, D), jnp.float32),
             pltpu.VMEM((BR, D), jnp.float32)]
            + [pltpu.VMEM((K, D), jnp.float32) for _ in range(2)]
            + [pltpu.VMEM((K, D), jnp.float32) for _ in range(2)]
            + [pltpu.VMEM((3 * K,), jnp.int32) for _ in range(2)]
            + [pltpu.VMEM((K,), jnp.int32) for _ in range(2)]
            + [pltpu.SemaphoreType.DMA for _ in range(6)]
        ),
    )
    return k(idx, ann, bias_row)


def kernel(edge_index, edge_vals, annotations, bias):
    dst = edge_index[:, 0, :].reshape(B, NS, CHUNKS, K)
    src = edge_index[:, 1, :].reshape(B, NS, CHUNKS, K)
    ev = lax.bitcast_convert_type(edge_vals, jnp.int32).reshape(
        B, NS, CHUNKS, K)
    idx = jnp.stack([src, dst, ev], axis=3).reshape(B * NS * CHUNKS * 3 * K)
    bias_row = bias.reshape(1, D)
    return _gcn(idx, annotations, bias_row)
